# baseline (speedup 1.0000x reference)
import functools

import jax
import jax.numpy as jnp
from jax import lax
from jax.experimental import pallas as pl
from jax.experimental.pallas import tpu as pltpu
from jax.experimental.pallas import tpu_sc as plsc

NC = 2
NS = 16
NW = NC * NS
LANES = 16
CHUNK = 1392
NCHUNKS = 36


def _lj_body(n_nodes, np_rows, tstart,
             p_hbm, i_hbm, j_hbm, eps_hbm, sig_hbm,
             it_hbm, jt_hbm, et_hbm, st_hbm,
             fpart_hbm, epart_hbm,
             sh_p, sh_fx, sh_fy, sh_fz,
             ii, jj, i3, j3, eps_v, sig_v,
             gx_i, gy_i, gz_i, gx_j, gy_j, gz_j,
             fx_i, fy_i, fz_i, fx_j, fy_j, fz_j,
             ev, sem_l, sem_g, sem_s):
    c = lax.axis_index("c")
    s = lax.axis_index("s")
    wid = c * NS + s

    rows = np_rows // NS
    r0 = s * rows
    pieces = []
    off = 0
    while off < rows:
        pieces.append((off, min(CHUNK, rows - off)))
        off += CHUNK
    bounce = gx_i[0]
    nel = rows * 3
    e0s = s * nel
    ppieces = []
    off = 0
    while off < nel:
        ppieces.append((off, min(CHUNK, nel - off)))
        off += CHUNK
    for (o, ln) in ppieces:
        pltpu.sync_copy(p_hbm.at[pl.ds(e0s + o, ln)], bounce.at[pl.ds(0, ln)])
        pltpu.sync_copy(bounce.at[pl.ds(0, ln)], sh_p.at[pl.ds(e0s + o, ln)])

    def zbuf(t, _):
        bounce[pl.ds(t * LANES, LANES)] = jnp.zeros((LANES,), jnp.float32)
        return 0
    lax.fori_loop(0, CHUNK // LANES, zbuf, 0)
    for sh_ref in (sh_fx, sh_fy, sh_fz):
        for (o, ln) in pieces:
            pltpu.sync_copy(bounce.at[pl.ds(0, ln)], sh_ref.at[pl.ds(r0 + o, ln)])
    plsc.subcore_barrier()

    ebase0 = wid * (NCHUNKS * CHUNK)

    def loads(k, b3, b2, fire):
        eb = ebase0 + k * CHUNK
        if fire:
            @pl.when(eb < tstart)
            def _():
                pltpu.async_copy(i_hbm.at[pl.ds(eb, CHUNK)], ii[b3], sem_l)
                pltpu.async_copy(j_hbm.at[pl.ds(eb, CHUNK)], jj[b3], sem_l)
                pltpu.async_copy(eps_hbm.at[pl.ds(eb, CHUNK)], eps_v[b2], sem_l)
                pltpu.async_copy(sig_hbm.at[pl.ds(eb, CHUNK)], sig_v[b2], sem_l)

            @pl.when(eb >= tstart)
            def _():
                tb = eb - tstart
                pltpu.async_copy(it_hbm.at[pl.ds(tb, CHUNK)], ii[b3], sem_l)
                pltpu.async_copy(jt_hbm.at[pl.ds(tb, CHUNK)], jj[b3], sem_l)
                pltpu.async_copy(et_hbm.at[pl.ds(tb, CHUNK)], eps_v[b2], sem_l)
                pltpu.async_copy(st_hbm.at[pl.ds(tb, CHUNK)], sig_v[b2], sem_l)
        else:
            pltpu.make_async_copy(i_hbm.at[pl.ds(0, CHUNK)], ii[b3], sem_l).wait()
            pltpu.make_async_copy(j_hbm.at[pl.ds(0, CHUNK)], jj[b3], sem_l).wait()
            pltpu.make_async_copy(eps_hbm.at[pl.ds(0, CHUNK)], eps_v[b2], sem_l).wait()
            pltpu.make_async_copy(sig_hbm.at[pl.ds(0, CHUNK)], sig_v[b2], sem_l).wait()

    def make_i3(b3, b2):
        iv, jv = ii[b3], jj[b3]
        i30, i31, i32 = i3[b2]
        j30, j31, j32 = j3[b2]

        def tr(t, _):
            vs = pl.ds(t * LANES, LANES)
            v = iv[vs] * 3
            i30[vs] = v
            i31[vs] = v + 1
            i32[vs] = v + 2
            w = jv[vs] * 3
            j30[vs] = w
            j31[vs] = w + 1
            j32[vs] = w + 2
            return 0
        lax.fori_loop(0, CHUNK // LANES, tr, 0)

    def gathers(b3, b2, fire):
        i30, i31, i32 = i3[b2]
        j30, j31, j32 = j3[b2]
        op = pltpu.async_copy if fire else pltpu.make_async_copy
        cps = [
            op(sh_p.at[i30], gx_i[b2], sem_g),
            op(sh_p.at[i31], gy_i[b2], sem_g),
            op(sh_p.at[i32], gz_i[b2], sem_g),
            op(sh_p.at[j30], gx_j[b2], sem_g),
            op(sh_p.at[j31], gy_j[b2], sem_g),
            op(sh_p.at[j32], gz_j[b2], sem_g),
        ]
        if not fire:
            for cp in cps:
                cp.wait()

    def scatters(b3, b2, fire):
        if fire:
            pltpu.async_copy(fx_i[b2], sh_fx.at[ii[b3]], sem_s, add=True)
            pltpu.async_copy(fy_i[b2], sh_fy.at[ii[b3]], sem_s, add=True)
            pltpu.async_copy(fz_i[b2], sh_fz.at[ii[b3]], sem_s, add=True)
            pltpu.async_copy(fx_j[b2], sh_fx.at[jj[b3]], sem_s, add=True)
            pltpu.async_copy(fy_j[b2], sh_fy.at[jj[b3]], sem_s, add=True)
            pltpu.async_copy(fz_j[b2], sh_fz.at[jj[b3]], sem_s, add=True)
        else:
            pltpu.make_async_copy(fx_i[b2], sh_fx.at[ii[b3]], sem_s).wait()
            pltpu.make_async_copy(fy_i[b2], sh_fy.at[ii[b3]], sem_s).wait()
            pltpu.make_async_copy(fz_i[b2], sh_fz.at[ii[b3]], sem_s).wait()
            pltpu.make_async_copy(fx_j[b2], sh_fx.at[jj[b3]], sem_s).wait()
            pltpu.make_async_copy(fy_j[b2], sh_fy.at[jj[b3]], sem_s).wait()
            pltpu.make_async_copy(fz_j[b2], sh_fz.at[jj[b3]], sem_s).wait()

    def compute(b, eacc):
        xi, yi, zi = gx_i[b], gy_i[b], gz_i[b]
        xj, yj, zj = gx_j[b], gy_j[b], gz_j[b]
        fxi, fyi, fzi = fx_i[b], fy_i[b], fz_i[b]
        fxj, fyj, fzj = fx_j[b], fy_j[b], fz_j[b]
        epsb, sigb = eps_v[b], sig_v[b]

        def grp(t, eacc):
            vs = pl.ds(t * LANES, LANES)
            dx = xi[vs] - xj[vs]
            dy = yi[vs] - yj[vs]
            dz = zi[vs] - zj[vs]
            dx = dx - jnp.where(dx > 15.0, 30.0, jnp.where(dx < -15.0, -30.0, 0.0))
            dy = dy - jnp.where(dy > 15.0, 30.0, jnp.where(dy < -15.0, -30.0, 0.0))
            dz = dz - jnp.where(dz > 15.0, 30.0, jnp.where(dz < -15.0, -30.0, 0.0))
            r2 = jnp.maximum(dx * dx + dy * dy + dz * dz, 1e-24)
            inv_r2 = 1.0 / r2
            ep = epsb[vs]
            sg = sigb[vs]
            s2 = sg * sg * inv_r2
            s6 = s2 * s2 * s2
            s12 = s6 * s6
            mask = r2 < 9.0
            u = jnp.where(mask, 4.0 * ep * (s12 - s6), 0.0)
            fsc = jnp.where(mask, 24.0 * ep * inv_r2 * (2.0 * s12 - s6), 0.0)
            fx = fsc * dx
            fy = fsc * dy
            fz = fsc * dz
            fxi[vs] = fx
            fyi[vs] = fy
            fzi[vs] = fz
            fxj[vs] = -fx
            fyj[vs] = -fy
            fzj[vs] = -fz
            return eacc + u

        return lax.fori_loop(0, CHUNK // LANES, grp, eacc)

    loads(0, 0, 0, fire=True)
    loads(0, 0, 0, fire=False)
    make_i3(0, 0)
    gathers(0, 0, fire=True)
    loads(1, 1, 1, fire=True)

    NT = NCHUNKS // 6

    def six_body(t, eacc):
        for b in range(6):
            k = 6 * t + b
            b2, b3 = b % 2, b % 3
            n2, n3 = (b + 1) % 2, (b + 1) % 3
            p2, p3 = (b + 5) % 2, (b + 5) % 3
            gathers(b3, b2, fire=False)
            eacc = compute(b2, eacc)
            if b > 0:
                scatters(p3, p2, fire=False)
            else:
                @pl.when(t > 0)
                def _():
                    scatters(p3, p2, fire=False)
            scatters(b3, b2, fire=True)
            if b < 5:
                loads(k + 1, n3, n2, fire=False)
                make_i3(n3, n2)
                gathers(n3, n2, fire=True)
            else:
                @pl.when(t < NT - 1)
                def _():
                    loads(k + 1, n3, n2, fire=False)
                    make_i3(n3, n2)
                    gathers(n3, n2, fire=True)
            if b < 4:
                loads(k + 2, (b + 2) % 3, b2, fire=True)
            else:
                @pl.when(t < NT - 1)
                def _():
                    loads(k + 2, (b + 2) % 3, b2, fire=True)
        return eacc

    eacc = lax.fori_loop(0, NT, six_body, jnp.zeros((LANES,), jnp.float32))
    scatters((NCHUNKS - 1) % 3, (NCHUNKS - 1) % 2, fire=False)

    plsc.subcore_barrier()
    base = c * 3 * np_rows
    for comp, sh_ref in enumerate((sh_fx, sh_fy, sh_fz)):
        for (o, ln) in pieces:
            pltpu.sync_copy(sh_ref.at[pl.ds(r0 + o, ln)], bounce.at[pl.ds(0, ln)])
            pltpu.sync_copy(bounce.at[pl.ds(0, ln)],
                            fpart_hbm.at[pl.ds(base + comp * np_rows + r0 + o, ln)])
    ev[...] = eacc
    pltpu.sync_copy(ev, epart_hbm.at[pl.ds(wid * LANES, LANES)])


@jax.jit
def kernel(pos, edge_index, epsilon, sigma):
    n = pos.shape[0]
    e = epsilon.shape[0]
    rows_per_sub = -(-n // (NS * 8)) * 8
    np_rows = rows_per_sub * NS
    epad = NW * CHUNK * NCHUNKS
    assert epad >= e, (epad, e)

    p_flat = jnp.pad(pos.reshape(-1), (0, (np_rows - n) * 3))
    pad = epad - e
    ntail = max(1, -(-pad // CHUNK))
    tstart = epad - ntail * CHUNK
    i_p = edge_index[0].astype(jnp.int32)
    j_p = edge_index[1].astype(jnp.int32)
    zi = jnp.zeros((pad,), jnp.int32)
    zf = jnp.zeros((pad,), jnp.float32)
    it = jnp.concatenate([i_p[tstart:], zi])
    jt = jnp.concatenate([j_p[tstart:], zi])
    et = jnp.concatenate([epsilon[tstart:], zf])
    st = jnp.concatenate([sigma[tstart:], zf])

    mesh = plsc.VectorSubcoreMesh(core_axis_name="c", subcore_axis_name="s")
    run = pl.kernel(
        functools.partial(_lj_body, n, np_rows, tstart),
        out_type=(
            jax.ShapeDtypeStruct((NC * 3 * np_rows,), jnp.float32),
            jax.ShapeDtypeStruct((NW * LANES,), jnp.float32),
        ),
        mesh=mesh,
        scratch_types=(
            [pltpu.VMEM_SHARED((np_rows * 3,), jnp.float32)]
            + [pltpu.VMEM_SHARED((np_rows,), jnp.float32) for _ in range(3)]
            + [[pltpu.VMEM((CHUNK,), jnp.int32) for _ in range(3)]
               for _ in range(2)]
            + [[[pltpu.VMEM((CHUNK,), jnp.int32) for _ in range(3)]
                for _ in range(2)]
               for _ in range(2)]
            + [[pltpu.VMEM((CHUNK,), jnp.float32) for _ in range(2)]
               for _ in range(14)]
            + [pltpu.VMEM((LANES,), jnp.float32),
               pltpu.SemaphoreType.DMA,
               pltpu.SemaphoreType.DMA,
               pltpu.SemaphoreType.DMA]
        ),
    )
    fpart, epart = run(p_flat, i_p, j_p, epsilon, sigma, it, jt, et, st)
    fp = fpart.reshape(NC, 3, np_rows)
    forces = (fp[0] + fp[1])[:, :n].T
    total_energy = jnp.sum(epart)
    return (total_energy, forces)


# speedup vs baseline: 1.0847x; 1.0847x over previous
import functools

import jax
import jax.numpy as jnp
from jax import lax
from jax.experimental import pallas as pl
from jax.experimental.pallas import tpu as pltpu
from jax.experimental.pallas import tpu_sc as plsc

NC = 2
NS = 16
NW = NC * NS
LANES = 16
CHUNK = 696
NCHUNKS = 72


def _lj_body(n_nodes, np_rows, tstart,
             px_hbm, py_hbm, pz_hbm, i_hbm, j_hbm, eps_hbm, sig_hbm,
             it_hbm, jt_hbm, et_hbm, st_hbm,
             fpart_hbm, epart_hbm,
             sh_x, sh_y, sh_z, sh_fx, sh_fy, sh_fz,
             ii, jj, eps_v, sig_v,
             gx_i, gy_i, gz_i, gx_j, gy_j, gz_j,
             fx_i, fy_i, fz_i, fx_j, fy_j, fz_j,
             ev, sem_l, sem_g, sem_s):
    c = lax.axis_index("c")
    s = lax.axis_index("s")
    wid = c * NS + s

    rows = np_rows // NS
    r0 = s * rows
    pieces = []
    off = 0
    while off < rows:
        pieces.append((off, min(CHUNK, rows - off)))
        off += CHUNK
    bounce = gx_i[0]
    for hbm_ref, sh_ref in ((px_hbm, sh_x), (py_hbm, sh_y), (pz_hbm, sh_z)):
        for (o, ln) in pieces:
            pltpu.sync_copy(hbm_ref.at[pl.ds(r0 + o, ln)], bounce.at[pl.ds(0, ln)])
            pltpu.sync_copy(bounce.at[pl.ds(0, ln)], sh_ref.at[pl.ds(r0 + o, ln)])

    def zbuf(t, _):
        bounce[pl.ds(t * LANES, LANES)] = jnp.zeros((LANES,), jnp.float32)
        return 0
    lax.fori_loop(0, CHUNK // LANES, zbuf, 0)
    for sh_ref in (sh_fx, sh_fy, sh_fz):
        for (o, ln) in pieces:
            pltpu.sync_copy(bounce.at[pl.ds(0, ln)], sh_ref.at[pl.ds(r0 + o, ln)])
    plsc.subcore_barrier()

    ebase0 = wid * (NCHUNKS * CHUNK)

    def loads(k, b3, b2, fire):
        eb = ebase0 + k * CHUNK
        if fire:
            @pl.when(eb < tstart)
            def _():
                pltpu.async_copy(i_hbm.at[pl.ds(eb, CHUNK)], ii[b3], sem_l)
                pltpu.async_copy(j_hbm.at[pl.ds(eb, CHUNK)], jj[b3], sem_l)
                pltpu.async_copy(eps_hbm.at[pl.ds(eb, CHUNK)], eps_v[b2], sem_l)
                pltpu.async_copy(sig_hbm.at[pl.ds(eb, CHUNK)], sig_v[b2], sem_l)

            @pl.when(eb >= tstart)
            def _():
                tb = eb - tstart
                pltpu.async_copy(it_hbm.at[pl.ds(tb, CHUNK)], ii[b3], sem_l)
                pltpu.async_copy(jt_hbm.at[pl.ds(tb, CHUNK)], jj[b3], sem_l)
                pltpu.async_copy(et_hbm.at[pl.ds(tb, CHUNK)], eps_v[b2], sem_l)
                pltpu.async_copy(st_hbm.at[pl.ds(tb, CHUNK)], sig_v[b2], sem_l)
        else:
            pltpu.make_async_copy(i_hbm.at[pl.ds(0, CHUNK)], ii[b3], sem_l).wait()
            pltpu.make_async_copy(j_hbm.at[pl.ds(0, CHUNK)], jj[b3], sem_l).wait()
            pltpu.make_async_copy(eps_hbm.at[pl.ds(0, CHUNK)], eps_v[b2], sem_l).wait()
            pltpu.make_async_copy(sig_hbm.at[pl.ds(0, CHUNK)], sig_v[b2], sem_l).wait()

    def gathers(b3, b2, fire):
        op = pltpu.async_copy if fire else pltpu.make_async_copy
        cps = [
            op(sh_x.at[ii[b3]], gx_i[b2], sem_g),
            op(sh_y.at[ii[b3]], gy_i[b2], sem_g),
            op(sh_z.at[ii[b3]], gz_i[b2], sem_g),
            op(sh_x.at[jj[b3]], gx_j[b2], sem_g),
            op(sh_y.at[jj[b3]], gy_j[b2], sem_g),
            op(sh_z.at[jj[b3]], gz_j[b2], sem_g),
        ]
        if not fire:
            for cp in cps:
                cp.wait()

    def scatters(b3, b2, fire):
        if fire:
            pltpu.async_copy(fx_i[b2], sh_fx.at[ii[b3]], sem_s, add=True)
            pltpu.async_copy(fy_i[b2], sh_fy.at[ii[b3]], sem_s, add=True)
            pltpu.async_copy(fz_i[b2], sh_fz.at[ii[b3]], sem_s, add=True)
            pltpu.async_copy(fx_j[b2], sh_fx.at[jj[b3]], sem_s, add=True)
            pltpu.async_copy(fy_j[b2], sh_fy.at[jj[b3]], sem_s, add=True)
            pltpu.async_copy(fz_j[b2], sh_fz.at[jj[b3]], sem_s, add=True)
        else:
            pltpu.make_async_copy(fx_i[b2], sh_fx.at[ii[b3]], sem_s).wait()
            pltpu.make_async_copy(fy_i[b2], sh_fy.at[ii[b3]], sem_s).wait()
            pltpu.make_async_copy(fz_i[b2], sh_fz.at[ii[b3]], sem_s).wait()
            pltpu.make_async_copy(fx_j[b2], sh_fx.at[jj[b3]], sem_s).wait()
            pltpu.make_async_copy(fy_j[b2], sh_fy.at[jj[b3]], sem_s).wait()
            pltpu.make_async_copy(fz_j[b2], sh_fz.at[jj[b3]], sem_s).wait()

    def compute(b, eacc):
        xi, yi, zi = gx_i[b], gy_i[b], gz_i[b]
        xj, yj, zj = gx_j[b], gy_j[b], gz_j[b]
        fxi, fyi, fzi = fx_i[b], fy_i[b], fz_i[b]
        fxj, fyj, fzj = fx_j[b], fy_j[b], fz_j[b]
        epsb, sigb = eps_v[b], sig_v[b]

        def grp(t, eacc):
            vs = pl.ds(t * LANES, LANES)
            dx = xi[vs] - xj[vs]
            dy = yi[vs] - yj[vs]
            dz = zi[vs] - zj[vs]
            dx = dx - jnp.where(dx > 15.0, 30.0, jnp.where(dx < -15.0, -30.0, 0.0))
            dy = dy - jnp.where(dy > 15.0, 30.0, jnp.where(dy < -15.0, -30.0, 0.0))
            dz = dz - jnp.where(dz > 15.0, 30.0, jnp.where(dz < -15.0, -30.0, 0.0))
            r2 = jnp.maximum(dx * dx + dy * dy + dz * dz, 1e-24)
            inv_r2 = 1.0 / r2
            ep = epsb[vs]
            sg = sigb[vs]
            s2 = sg * sg * inv_r2
            s6 = s2 * s2 * s2
            s12 = s6 * s6
            mask = r2 < 9.0
            u = jnp.where(mask, 4.0 * ep * (s12 - s6), 0.0)
            fsc = jnp.where(mask, 24.0 * ep * inv_r2 * (2.0 * s12 - s6), 0.0)
            fx = fsc * dx
            fy = fsc * dy
            fz = fsc * dz
            fxi[vs] = fx
            fyi[vs] = fy
            fzi[vs] = fz
            fxj[vs] = -fx
            fyj[vs] = -fy
            fzj[vs] = -fz
            return eacc + u

        return lax.fori_loop(0, CHUNK // LANES, grp, eacc)

    loads(0, 0, 0, fire=True)
    loads(0, 0, 0, fire=False)
    gathers(0, 0, fire=True)
    loads(1, 1, 1, fire=True)

    NT = NCHUNKS // 6

    def six_body(t, eacc):
        for b in range(6):
            k = 6 * t + b
            b2, b3 = b % 2, b % 3
            n2, n3 = (b + 1) % 2, (b + 1) % 3
            p2, p3 = (b + 5) % 2, (b + 5) % 3
            gathers(b3, b2, fire=False)
            eacc = compute(b2, eacc)
            if b > 0:
                scatters(p3, p2, fire=False)
            else:
                @pl.when(t > 0)
                def _():
                    scatters(p3, p2, fire=False)
            scatters(b3, b2, fire=True)
            if b < 5:
                loads(k + 1, n3, n2, fire=False)
                gathers(n3, n2, fire=True)
            else:
                @pl.when(t < NT - 1)
                def _():
                    loads(k + 1, n3, n2, fire=False)
                    gathers(n3, n2, fire=True)
            if b < 4:
                loads(k + 2, (b + 2) % 3, b2, fire=True)
            else:
                @pl.when(t < NT - 1)
                def _():
                    loads(k + 2, (b + 2) % 3, b2, fire=True)
        return eacc

    eacc = lax.fori_loop(0, NT, six_body, jnp.zeros((LANES,), jnp.float32))
    scatters((NCHUNKS - 1) % 3, (NCHUNKS - 1) % 2, fire=False)

    plsc.subcore_barrier()
    base = c * 3 * np_rows
    for comp, sh_ref in enumerate((sh_fx, sh_fy, sh_fz)):
        for (o, ln) in pieces:
            pltpu.sync_copy(sh_ref.at[pl.ds(r0 + o, ln)], bounce.at[pl.ds(0, ln)])
            pltpu.sync_copy(bounce.at[pl.ds(0, ln)],
                            fpart_hbm.at[pl.ds(base + comp * np_rows + r0 + o, ln)])
    ev[...] = eacc
    pltpu.sync_copy(ev, epart_hbm.at[pl.ds(wid * LANES, LANES)])


@jax.jit
def kernel(pos, edge_index, epsilon, sigma):
    n = pos.shape[0]
    e = epsilon.shape[0]
    rows_per_sub = -(-n // (NS * 8)) * 8
    np_rows = rows_per_sub * NS
    epad = NW * CHUNK * NCHUNKS
    assert epad >= e, (epad, e)

    pz3 = jnp.zeros((np_rows - n,), jnp.float32)
    px = jnp.concatenate([pos[:, 0], pz3])
    py = jnp.concatenate([pos[:, 1], pz3])
    pz = jnp.concatenate([pos[:, 2], pz3])
    pad = epad - e
    ntail = max(1, -(-pad // CHUNK))
    tstart = epad - ntail * CHUNK
    i_p = edge_index[0].astype(jnp.int32)
    j_p = edge_index[1].astype(jnp.int32)
    zi = jnp.zeros((pad,), jnp.int32)
    zf = jnp.zeros((pad,), jnp.float32)
    it = jnp.concatenate([i_p[tstart:], zi])
    jt = jnp.concatenate([j_p[tstart:], zi])
    et = jnp.concatenate([epsilon[tstart:], zf])
    st = jnp.concatenate([sigma[tstart:], zf])

    mesh = plsc.VectorSubcoreMesh(core_axis_name="c", subcore_axis_name="s")
    run = pl.kernel(
        functools.partial(_lj_body, n, np_rows, tstart),
        out_type=(
            jax.ShapeDtypeStruct((NC * 3 * np_rows,), jnp.float32),
            jax.ShapeDtypeStruct((NW * LANES,), jnp.float32),
        ),
        mesh=mesh,
        scratch_types=(
            [pltpu.VMEM_SHARED((np_rows,), jnp.float32) for _ in range(6)]
            + [[pltpu.VMEM((CHUNK,), jnp.int32) for _ in range(3)]
               for _ in range(2)]
            + [[pltpu.VMEM((CHUNK,), jnp.float32) for _ in range(2)]
               for _ in range(14)]
            + [pltpu.VMEM((LANES,), jnp.float32),
               pltpu.SemaphoreType.DMA,
               pltpu.SemaphoreType.DMA,
               pltpu.SemaphoreType.DMA]
        ),
    )
    fpart, epart = run(px, py, pz, i_p, j_p, epsilon, sigma, it, jt, et, st)
    fp = fpart.reshape(NC, 3, np_rows)
    forces = (fp[0] + fp[1])[:, :n].T
    total_energy = jnp.sum(epart)
    return (total_energy, forces)


# speedup vs baseline: 1.1816x; 1.0893x over previous
import functools

import jax
import jax.numpy as jnp
from jax import lax
from jax.experimental import pallas as pl
from jax.experimental.pallas import tpu as pltpu
from jax.experimental.pallas import tpu_sc as plsc

NC = 2
NS = 16
NW = NC * NS
LANES = 16
CHUNK = 1392
NCHUNKS = 36


def _lj_body(n_nodes, np_rows, tstart,
             px_hbm, py_hbm, pz_hbm, i_hbm, j_hbm, eps_hbm, sig_hbm,
             it_hbm, jt_hbm, et_hbm, st_hbm,
             fpart_hbm, epart_hbm,
             sh_x, sh_y, sh_z, sh_fx, sh_fy, sh_fz,
             ii, jj, eps_v, sig_v,
             gx_i, gy_i, gz_i, gx_j, gy_j, gz_j,
             fx_i, fy_i, fz_i, fx_j, fy_j, fz_j,
             ev, sem_l, sem_g, sem_s):
    c = lax.axis_index("c")
    s = lax.axis_index("s")
    wid = c * NS + s

    rows = np_rows // NS
    r0 = s * rows
    pieces = []
    off = 0
    while off < rows:
        pieces.append((off, min(CHUNK, rows - off)))
        off += CHUNK
    bounce = gx_i[0]
    for hbm_ref, sh_ref in ((px_hbm, sh_x), (py_hbm, sh_y), (pz_hbm, sh_z)):
        for (o, ln) in pieces:
            pltpu.sync_copy(hbm_ref.at[pl.ds(r0 + o, ln)], bounce.at[pl.ds(0, ln)])
            pltpu.sync_copy(bounce.at[pl.ds(0, ln)], sh_ref.at[pl.ds(r0 + o, ln)])

    def zbuf(t, _):
        bounce[pl.ds(t * LANES, LANES)] = jnp.zeros((LANES,), jnp.float32)
        return 0
    lax.fori_loop(0, CHUNK // LANES, zbuf, 0)
    for sh_ref in (sh_fx, sh_fy, sh_fz):
        for (o, ln) in pieces:
            pltpu.sync_copy(bounce.at[pl.ds(0, ln)], sh_ref.at[pl.ds(r0 + o, ln)])
    plsc.subcore_barrier()

    ebase0 = wid * (NCHUNKS * CHUNK)

    def loads(k, b3, b2, fire):
        eb = ebase0 + k * CHUNK
        if fire:
            @pl.when(eb < tstart)
            def _():
                pltpu.async_copy(i_hbm.at[pl.ds(eb, CHUNK)], ii[b3], sem_l)
                pltpu.async_copy(j_hbm.at[pl.ds(eb, CHUNK)], jj[b3], sem_l)
                pltpu.async_copy(eps_hbm.at[pl.ds(eb, CHUNK)], eps_v[b2], sem_l)
                pltpu.async_copy(sig_hbm.at[pl.ds(eb, CHUNK)], sig_v[b2], sem_l)

            @pl.when(eb >= tstart)
            def _():
                tb = eb - tstart
                pltpu.async_copy(it_hbm.at[pl.ds(tb, CHUNK)], ii[b3], sem_l)
                pltpu.async_copy(jt_hbm.at[pl.ds(tb, CHUNK)], jj[b3], sem_l)
                pltpu.async_copy(et_hbm.at[pl.ds(tb, CHUNK)], eps_v[b2], sem_l)
                pltpu.async_copy(st_hbm.at[pl.ds(tb, CHUNK)], sig_v[b2], sem_l)
        else:
            pltpu.make_async_copy(i_hbm.at[pl.ds(0, CHUNK)], ii[b3], sem_l).wait()
            pltpu.make_async_copy(j_hbm.at[pl.ds(0, CHUNK)], jj[b3], sem_l).wait()
            pltpu.make_async_copy(eps_hbm.at[pl.ds(0, CHUNK)], eps_v[b2], sem_l).wait()
            pltpu.make_async_copy(sig_hbm.at[pl.ds(0, CHUNK)], sig_v[b2], sem_l).wait()

    def gathers(b3, b2, fire):
        op = pltpu.async_copy if fire else pltpu.make_async_copy
        cps = [
            op(sh_x.at[ii[b3]], gx_i[b2], sem_g),
            op(sh_y.at[ii[b3]], gy_i[b2], sem_g),
            op(sh_z.at[ii[b3]], gz_i[b2], sem_g),
            op(sh_x.at[jj[b3]], gx_j[b2], sem_g),
            op(sh_y.at[jj[b3]], gy_j[b2], sem_g),
            op(sh_z.at[jj[b3]], gz_j[b2], sem_g),
        ]
        if not fire:
            for cp in cps:
                cp.wait()

    def scatters(b3, b2, fire):
        if fire:
            pltpu.async_copy(fx_i[b2], sh_fx.at[ii[b3]], sem_s, add=True)
            pltpu.async_copy(fy_i[b2], sh_fy.at[ii[b3]], sem_s, add=True)
            pltpu.async_copy(fz_i[b2], sh_fz.at[ii[b3]], sem_s, add=True)
            pltpu.async_copy(fx_j[b2], sh_fx.at[jj[b3]], sem_s, add=True)
            pltpu.async_copy(fy_j[b2], sh_fy.at[jj[b3]], sem_s, add=True)
            pltpu.async_copy(fz_j[b2], sh_fz.at[jj[b3]], sem_s, add=True)
        else:
            pltpu.make_async_copy(fx_i[b2], sh_fx.at[ii[b3]], sem_s).wait()
            pltpu.make_async_copy(fy_i[b2], sh_fy.at[ii[b3]], sem_s).wait()
            pltpu.make_async_copy(fz_i[b2], sh_fz.at[ii[b3]], sem_s).wait()
            pltpu.make_async_copy(fx_j[b2], sh_fx.at[jj[b3]], sem_s).wait()
            pltpu.make_async_copy(fy_j[b2], sh_fy.at[jj[b3]], sem_s).wait()
            pltpu.make_async_copy(fz_j[b2], sh_fz.at[jj[b3]], sem_s).wait()

    def compute(b, eacc):
        xi, yi, zi = gx_i[b], gy_i[b], gz_i[b]
        xj, yj, zj = gx_j[b], gy_j[b], gz_j[b]
        fxi, fyi, fzi = fx_i[b], fy_i[b], fz_i[b]
        fxj, fyj, fzj = fx_j[b], fy_j[b], fz_j[b]
        epsb, sigb = eps_v[b], sig_v[b]

        def grp(t, eacc):
            vs = pl.ds(t * LANES, LANES)
            dx = xi[vs] - xj[vs]
            dy = yi[vs] - yj[vs]
            dz = zi[vs] - zj[vs]
            dx = dx - jnp.where(dx > 15.0, 30.0, jnp.where(dx < -15.0, -30.0, 0.0))
            dy = dy - jnp.where(dy > 15.0, 30.0, jnp.where(dy < -15.0, -30.0, 0.0))
            dz = dz - jnp.where(dz > 15.0, 30.0, jnp.where(dz < -15.0, -30.0, 0.0))
            r2 = jnp.maximum(dx * dx + dy * dy + dz * dz, 1e-24)
            inv_r2 = 1.0 / r2
            ep = epsb[vs]
            sg = sigb[vs]
            s2 = sg * sg * inv_r2
            s6 = s2 * s2 * s2
            s12 = s6 * s6
            mask = r2 < 9.0
            u = jnp.where(mask, 4.0 * ep * (s12 - s6), 0.0)
            fsc = jnp.where(mask, 24.0 * ep * inv_r2 * (2.0 * s12 - s6), 0.0)
            fx = fsc * dx
            fy = fsc * dy
            fz = fsc * dz
            fxi[vs] = fx
            fyi[vs] = fy
            fzi[vs] = fz
            fxj[vs] = -fx
            fyj[vs] = -fy
            fzj[vs] = -fz
            return eacc + u

        return lax.fori_loop(0, CHUNK // LANES, grp, eacc)

    loads(0, 0, 0, fire=True)
    loads(0, 0, 0, fire=False)
    gathers(0, 0, fire=True)
    loads(1, 1, 1, fire=True)

    NT = NCHUNKS // 6

    def six_body(t, eacc):
        for b in range(6):
            k = 6 * t + b
            b2, b3 = b % 2, b % 3
            n2, n3 = (b + 1) % 2, (b + 1) % 3
            p2, p3 = (b + 5) % 2, (b + 5) % 3
            gathers(b3, b2, fire=False)
            if b < 5:
                loads(k + 1, n3, n2, fire=False)
                gathers(n3, n2, fire=True)
            else:
                @pl.when(t < NT - 1)
                def _():
                    loads(k + 1, n3, n2, fire=False)
                    gathers(n3, n2, fire=True)
            eacc = compute(b2, eacc)
            if b > 0:
                scatters(p3, p2, fire=False)
            else:
                @pl.when(t > 0)
                def _():
                    scatters(p3, p2, fire=False)
            scatters(b3, b2, fire=True)
            if b < 4:
                loads(k + 2, (b + 2) % 3, b2, fire=True)
            else:
                @pl.when(t < NT - 1)
                def _():
                    loads(k + 2, (b + 2) % 3, b2, fire=True)
        return eacc

    eacc = lax.fori_loop(0, NT, six_body, jnp.zeros((LANES,), jnp.float32))
    scatters((NCHUNKS - 1) % 3, (NCHUNKS - 1) % 2, fire=False)

    plsc.subcore_barrier()
    base = c * 3 * np_rows
    for comp, sh_ref in enumerate((sh_fx, sh_fy, sh_fz)):
        for (o, ln) in pieces:
            pltpu.sync_copy(sh_ref.at[pl.ds(r0 + o, ln)], bounce.at[pl.ds(0, ln)])
            pltpu.sync_copy(bounce.at[pl.ds(0, ln)],
                            fpart_hbm.at[pl.ds(base + comp * np_rows + r0 + o, ln)])
    ev[...] = eacc
    pltpu.sync_copy(ev, epart_hbm.at[pl.ds(wid * LANES, LANES)])


@jax.jit
def kernel(pos, edge_index, epsilon, sigma):
    n = pos.shape[0]
    e = epsilon.shape[0]
    rows_per_sub = -(-n // (NS * 8)) * 8
    np_rows = rows_per_sub * NS
    epad = NW * CHUNK * NCHUNKS
    assert epad >= e, (epad, e)

    pz3 = jnp.zeros((np_rows - n,), jnp.float32)
    px = jnp.concatenate([pos[:, 0], pz3])
    py = jnp.concatenate([pos[:, 1], pz3])
    pz = jnp.concatenate([pos[:, 2], pz3])
    pad = epad - e
    ntail = max(1, -(-pad // CHUNK))
    tstart = epad - ntail * CHUNK
    i_p = edge_index[0].astype(jnp.int32)
    j_p = edge_index[1].astype(jnp.int32)
    zi = jnp.zeros((pad,), jnp.int32)
    zf = jnp.zeros((pad,), jnp.float32)
    it = jnp.concatenate([i_p[tstart:], zi])
    jt = jnp.concatenate([j_p[tstart:], zi])
    et = jnp.concatenate([epsilon[tstart:], zf])
    st = jnp.concatenate([sigma[tstart:], zf])

    mesh = plsc.VectorSubcoreMesh(core_axis_name="c", subcore_axis_name="s")
    run = pl.kernel(
        functools.partial(_lj_body, n, np_rows, tstart),
        out_type=(
            jax.ShapeDtypeStruct((NC * 3 * np_rows,), jnp.float32),
            jax.ShapeDtypeStruct((NW * LANES,), jnp.float32),
        ),
        mesh=mesh,
        scratch_types=(
            [pltpu.VMEM_SHARED((np_rows,), jnp.float32) for _ in range(6)]
            + [[pltpu.VMEM((CHUNK,), jnp.int32) for _ in range(3)]
               for _ in range(2)]
            + [[pltpu.VMEM((CHUNK,), jnp.float32) for _ in range(2)]
               for _ in range(14)]
            + [pltpu.VMEM((LANES,), jnp.float32),
               pltpu.SemaphoreType.DMA,
               pltpu.SemaphoreType.DMA,
               pltpu.SemaphoreType.DMA]
        ),
    )
    fpart, epart = run(px, py, pz, i_p, j_p, epsilon, sigma, it, jt, et, st)
    fp = fpart.reshape(NC, 3, np_rows)
    forces = (fp[0] + fp[1])[:, :n].T
    total_energy = jnp.sum(epart)
    return (total_energy, forces)
